# baseline (speedup 1.0000x reference)
import functools

import jax
import jax.numpy as jnp
import numpy as np
from jax import lax
from jax.experimental import pallas as pl
from jax.experimental.pallas import tpu as pltpu
from jax.experimental.pallas import tpu_sc as plsc

_N_STATES = 16
_N_SYMBOLS = 8
_SEQ_LEN = 200
_LANES = 16
_NCHUNK = (_SEQ_LEN + _LANES - 1) // _LANES
_PADDED = _NCHUNK * _LANES
_ZROW = 33

_DFA = {0: {'0': 1, '1': 2, 'and(2,3)': 3}, 1: {'2': 3, 'not(0)': 0, '4': 5}, 2: {'or(1,5)': 4, '3': 2}, 3: {'5': 6, 'T': 0}, 4: {'6': 7, 'and(0,not(1))': 8}, 5: {'7': 9, '2': 5}, 6: {'or(and(0,1),2)': 10, '4': 6}, 7: {'1': 11, 'not(6)': 7}, 8: {'3': 12, '0': 8}, 9: {'5': 13, 'or(2,3)': 9}, 10: {'and(4,5)': 14, '6': 10}, 11: {'7': 15, '1': 11}, 12: {'0': 0, 'not(7)': 12}, 13: {'2': 1, '6': 13}, 14: {'or(0,not(4))': 2, '3': 14}, 15: {'T': 3}}

_TRANS = [(s, g, d) for s in sorted(_DFA.keys()) for g, d in _DFA[s].items()]

_INC = {}
for _t, (_s, _g, _d) in enumerate(_TRANS):
    _INC.setdefault(_d, []).append((_t, _s))
_N_GROUPS = max(len(v) for v in _INC.values())
_TID = np.full((_N_GROUPS, _N_STATES), _ZROW, np.int32)
_SRC = np.zeros((_N_GROUPS, _N_STATES), np.int32)
for _d in range(_N_STATES):
    for _j, (_t, _s) in enumerate(_INC[_d]):
        _TID[_j, _d] = _t
        _SRC[_j, _d] = _s


def _divide_args(guard):
    args = guard.split(',')
    out = []
    i = 0
    while i < len(args):
        a = args[i]
        while a.count('(') != a.count(')'):
            i += 1
            a = a + ',' + args[i]
        out.append(a)
        i += 1
    return out


def _eval_guard(guard, cols):
    if guard[0] == 'a':
        v = 1.0
        for a in _divide_args(guard[4:-1]):
            v = v * _eval_guard(a, cols)
        return v
    elif guard[0] == 'o':
        v = 0.0
        for a in _divide_args(guard[3:-1]):
            e = _eval_guard(a, cols)
            v = v + e - v * e
        return v
    elif guard[0] == 'n':
        return 1.0 - _eval_guard(guard[4:-1], cols)
    elif guard[0] == 'T':
        return jnp.ones_like(cols[0])
    else:
        return cols[int(guard)]


_GATHER_DNUMS = lax.GatherDimensionNumbers(
    offset_dims=(), collapsed_slice_dims=(0,), start_index_map=(0,))


def _vgather(x, idx):
    return lax.gather(x, idx[:, None], _GATHER_DNUMS, (1,),
                      mode=lax.GatherScatterMode.PROMISE_IN_BOUNDS)


def _sc_body(p_hbm, idx_hbm, out_hbm, p_v, idx_v, g_v, out_v):
    wid = lax.axis_index("s") * 2 + lax.axis_index("c")

    @pl.when(wid == 0)
    def _():
        pltpu.sync_copy(p_hbm, p_v.at[pl.ds(0, _SEQ_LEN)])
        pltpu.sync_copy(idx_hbm, idx_v)

        lane = lax.broadcasted_iota(jnp.int32, (_LANES,), 0)
        for c in range(_NCHUNK):
            ids = lane + (c * _LANES)
            cols = [plsc.load_gather(p_v, [ids, jnp.full((_LANES,), k, jnp.int32)])
                    for k in range(_N_SYMBOLS)]
            for t, (_, g, _2) in enumerate(_TRANS):
                g_v[t, pl.ds(c * _LANES, _LANES)] = _eval_guard(g, cols)
            g_v[_ZROW, pl.ds(c * _LANES, _LANES)] = jnp.zeros((_LANES,), jnp.float32)

        tids = [idx_v[j, :] for j in range(_N_GROUPS)]
        srcs = [idx_v[_N_GROUPS + j, :] for j in range(_N_GROUPS)]
        st0 = (lane == 0).astype(jnp.float32)

        def body(i, st):
            ii = jnp.full((_LANES,), i, jnp.int32)
            nxt = None
            for j in range(_N_GROUPS):
                gj = plsc.load_gather(g_v, [tids[j], ii])
                term = _vgather(st, srcs[j]) * gj
                nxt = term if nxt is None else nxt + term
            return nxt

        st = lax.fori_loop(0, _SEQ_LEN, body, st0)
        out_v[...] = st
        pltpu.sync_copy(out_v, out_hbm)


def kernel(symbols_prob):
    mesh = plsc.VectorSubcoreMesh(core_axis_name="c", subcore_axis_name="s")
    run = pl.kernel(
        _sc_body, mesh=mesh,
        out_type=jax.ShapeDtypeStruct((_N_STATES,), jnp.float32),
        compiler_params=pltpu.CompilerParams(needs_layout_passes=False),
        scratch_types=[
            pltpu.VMEM((_PADDED, _N_SYMBOLS), jnp.float32),
            pltpu.VMEM((2 * _N_GROUPS, _LANES), jnp.int32),
            pltpu.VMEM((_ZROW + 1, _PADDED), jnp.float32),
            pltpu.VMEM((_N_STATES,), jnp.float32),
        ],
    )
    idx_tab = jnp.asarray(np.concatenate([_TID, _SRC], axis=0))
    return run(symbols_prob, idx_tab)
